# baseline (speedup 1.0000x reference)
import functools

import jax
import jax.numpy as jnp
from jax import lax
from jax.experimental import pallas as pl
from jax.experimental.pallas import tpu as pltpu
from jax.experimental.pallas import tpu_sc as plsc

_N_CAND = 8
_ROWS = 2 * 2048
_D = 1024
_NW = 32
_RPW = _ROWS // _NW
_CH = 32
_NCH = _RPW // _CH
_NBUF = 3

_sc_mesh = plsc.VectorSubcoreMesh(core_axis_name="c", subcore_axis_name="s")


@functools.partial(
    pl.kernel,
    out_type=jax.ShapeDtypeStruct((_ROWS, _D), jnp.float32),
    mesh=_sc_mesh,
    scratch_types=[
        pltpu.VMEM((16,), jnp.int32),
        [pltpu.VMEM((_CH, _D), jnp.float32) for _ in range(_NBUF)],
        [pltpu.SemaphoreType.DMA for _ in range(_NBUF)],
        [pltpu.SemaphoreType.DMA for _ in range(_NBUF)],
    ],
)
def _sc_copy(table_hbm, s_hbm, out_hbm, s_vmem, bufs, gsems, ssems):
    wid = lax.axis_index("s") * 2 + lax.axis_index("c")
    base = wid * _RPW
    pltpu.sync_copy(s_hbm, s_vmem)
    src0 = s_vmem[...][0] * _ROWS + base

    def g(ch):
        return pltpu.make_async_copy(
            table_hbm.at[pl.ds(src0 + ch * _CH, _CH), :],
            bufs[ch % _NBUF],
            gsems[ch % _NBUF],
        )

    def s(ch):
        return pltpu.make_async_copy(
            bufs[ch % _NBUF],
            out_hbm.at[pl.ds(base + ch * _CH, _CH), :],
            ssems[ch % _NBUF],
        )

    for ch in range(min(_NBUF, _NCH)):
        g(ch).start()
    for ch in range(_NCH):
        g(ch).wait()
        s(ch).start()
        nxt = ch + _NBUF
        if nxt < _NCH:
            s(ch).wait()
            g(nxt).start()
    for ch in range(max(0, _NCH - _NBUF), _NCH):
        s(ch).wait()


def kernel(inputs, binary_gates, alpha, sampled):
    del binary_gates, alpha
    s = jnp.full((16,), sampled, dtype=jnp.int32)
    table = inputs.reshape(_N_CAND * _ROWS, _D)
    out = _sc_copy(table, s)
    return out.reshape(2, 2048, _D)


# speedup vs baseline: 2.0925x; 2.0925x over previous
import jax
import jax.numpy as jnp
from jax.experimental import pallas as pl
from jax.experimental.pallas import tpu as pltpu

_N_CAND = 8
_ROWS = 2 * 2048
_D = 1024
_NCHUNKS = 32
_CHUNK = _ROWS // _NCHUNKS
_WIN = 4


def _dma_body(s_ref, in_ref, out_ref, buf, sin, sout):
    s = s_ref[0]

    def ic(i):
        return pltpu.make_async_copy(
            in_ref.at[s, pl.ds(i * _CHUNK, _CHUNK), :],
            buf.at[pl.ds(i * _CHUNK, _CHUNK), :],
            sin.at[i],
        )

    def oc(i):
        return pltpu.make_async_copy(
            buf.at[pl.ds(i * _CHUNK, _CHUNK), :],
            out_ref.at[pl.ds(i * _CHUNK, _CHUNK), :],
            sout.at[i],
        )

    for i in range(_WIN):
        ic(i).start()
    for i in range(_NCHUNKS):
        ic(i).wait()
        oc(i).start()
        if i + _WIN < _NCHUNKS:
            ic(i + _WIN).start()
    for i in range(_NCHUNKS):
        oc(i).wait()


def kernel(inputs, binary_gates, alpha, sampled):
    del binary_gates, alpha
    s = jnp.asarray(sampled, dtype=jnp.int32).reshape((1,))
    flat = inputs.reshape(_N_CAND, _ROWS, _D)
    out = pl.pallas_call(
        _dma_body,
        grid_spec=pltpu.PrefetchScalarGridSpec(
            num_scalar_prefetch=1,
            in_specs=[pl.BlockSpec(memory_space=pl.ANY)],
            out_specs=pl.BlockSpec(memory_space=pl.ANY),
            scratch_shapes=[
                pltpu.VMEM((_ROWS, _D), jnp.float32),
                pltpu.SemaphoreType.DMA((_NCHUNKS,)),
                pltpu.SemaphoreType.DMA((_NCHUNKS,)),
            ],
        ),
        out_shape=jax.ShapeDtypeStruct((_ROWS, _D), jnp.float32),
    )(s, flat)
    return out.reshape(2, 2048, _D)


# speedup vs baseline: 2.6636x; 1.2730x over previous
import jax
import jax.numpy as jnp
from jax.experimental import pallas as pl
from jax.experimental.pallas import tpu as pltpu

_N_CAND = 8
_ROWS = 2 * 2048
_D = 1024
_NCHUNKS = 16
_CHUNK = _ROWS // _NCHUNKS
_WIN = 16


def _dma_body(s_ref, in_ref, out_ref, buf, sin, sout):
    s = s_ref[0]

    def ic(i):
        return pltpu.make_async_copy(
            in_ref.at[s, pl.ds(i * _CHUNK, _CHUNK), :],
            buf.at[pl.ds(i * _CHUNK, _CHUNK), :],
            sin.at[i],
        )

    def oc(i):
        return pltpu.make_async_copy(
            buf.at[pl.ds(i * _CHUNK, _CHUNK), :],
            out_ref.at[pl.ds(i * _CHUNK, _CHUNK), :],
            sout.at[i],
        )

    for i in range(_WIN):
        ic(i).start()
    for i in range(_NCHUNKS):
        ic(i).wait()
        oc(i).start()
        if i + _WIN < _NCHUNKS:
            ic(i + _WIN).start()
    for i in range(_NCHUNKS):
        oc(i).wait()


def kernel(inputs, binary_gates, alpha, sampled):
    del binary_gates, alpha
    s = jnp.asarray(sampled, dtype=jnp.int32).reshape((1,))
    flat = inputs.reshape(_N_CAND, _ROWS, _D)
    out = pl.pallas_call(
        _dma_body,
        grid_spec=pltpu.PrefetchScalarGridSpec(
            num_scalar_prefetch=1,
            in_specs=[pl.BlockSpec(memory_space=pl.ANY)],
            out_specs=pl.BlockSpec(memory_space=pl.ANY),
            scratch_shapes=[
                pltpu.VMEM((_ROWS, _D), jnp.float32),
                pltpu.SemaphoreType.DMA((_NCHUNKS,)),
                pltpu.SemaphoreType.DMA((_NCHUNKS,)),
            ],
        ),
        out_shape=jax.ShapeDtypeStruct((_ROWS, _D), jnp.float32),
    )(s, flat)
    return out.reshape(2, 2048, _D)


# speedup vs baseline: 2.7598x; 1.0361x over previous
import jax
import jax.numpy as jnp
from jax.experimental import pallas as pl
from jax.experimental.pallas import tpu as pltpu

_N_CAND = 8
_ROWS = 2 * 2048
_D = 1024
_NCHUNKS = 8
_CHUNK = _ROWS // _NCHUNKS
_WIN = 8


def _dma_body(s_ref, in_ref, out_ref, buf, sin, sout):
    s = s_ref[0]

    def ic(i):
        return pltpu.make_async_copy(
            in_ref.at[s, pl.ds(i * _CHUNK, _CHUNK), :],
            buf.at[pl.ds(i * _CHUNK, _CHUNK), :],
            sin.at[i],
        )

    def oc(i):
        return pltpu.make_async_copy(
            buf.at[pl.ds(i * _CHUNK, _CHUNK), :],
            out_ref.at[pl.ds(i * _CHUNK, _CHUNK), :],
            sout.at[i],
        )

    for i in range(_WIN):
        ic(i).start()
    for i in range(_NCHUNKS):
        ic(i).wait()
        oc(i).start()
        if i + _WIN < _NCHUNKS:
            ic(i + _WIN).start()
    for i in range(_NCHUNKS):
        oc(i).wait()


def kernel(inputs, binary_gates, alpha, sampled):
    del binary_gates, alpha
    s = jnp.asarray(sampled, dtype=jnp.int32).reshape((1,))
    flat = inputs.reshape(_N_CAND, _ROWS, _D)
    out = pl.pallas_call(
        _dma_body,
        grid_spec=pltpu.PrefetchScalarGridSpec(
            num_scalar_prefetch=1,
            in_specs=[pl.BlockSpec(memory_space=pl.ANY)],
            out_specs=pl.BlockSpec(memory_space=pl.ANY),
            scratch_shapes=[
                pltpu.VMEM((_ROWS, _D), jnp.float32),
                pltpu.SemaphoreType.DMA((_NCHUNKS,)),
                pltpu.SemaphoreType.DMA((_NCHUNKS,)),
            ],
        ),
        out_shape=jax.ShapeDtypeStruct((_ROWS, _D), jnp.float32),
    )(s, flat)
    return out.reshape(2, 2048, _D)
